# baseline (speedup 1.0000x reference)
import jax
import jax.numpy as jnp
from jax import lax
from jax.experimental import pallas as pl
from jax.experimental.pallas import tpu as pltpu
from jax.experimental.pallas import tpu_sc as plsc

D = 128
EPS = 1e-12
NW = 32
CHUNK = 64
NBUF = 10
LN_BLK = 10000


def _ln_table_body(x_ref, w_ref, b_ref, o_ref):
    x = x_ref[...]
    mean = jnp.mean(x, axis=1, keepdims=True)
    xc = x - mean
    var = jnp.mean(xc * xc, axis=1, keepdims=True)
    o_ref[...] = xc * lax.rsqrt(var + EPS) * w_ref[...] + b_ref[...]


def _gather_body(ids_hbm, table_hbm, out_hbm, idx_v, *rest):
    bufs = rest[:NBUF]
    sg = rest[NBUF:2 * NBUF]
    ss = rest[2 * NBUF:3 * NBUF]
    npw = ids_hbm.shape[0] // NW
    ncw = npw // CHUNK
    ngroups = ncw // NBUF
    wid = lax.axis_index("s") * 2 + lax.axis_index("c")
    rbase = pl.multiple_of(wid * npw, CHUNK)
    pltpu.sync_copy(ids_hbm.at[pl.ds(rbase, npw)], idx_v)

    def start_gather(j, buf, sem):
        coff = pl.multiple_of(j * CHUNK, CHUNK)
        pltpu.async_copy(table_hbm.at[idx_v.at[pl.ds(coff, CHUNK)]], buf, sem)

    def wait_gather(buf, sem):
        pltpu.make_async_copy(table_hbm.at[pl.ds(0, CHUNK)], buf, sem).wait()

    def start_store(j, buf, sem):
        ooff = pl.multiple_of(rbase + j * CHUNK, CHUNK)
        pltpu.async_copy(buf, out_hbm.at[pl.ds(ooff, CHUNK)], sem)

    def wait_store(buf, sem):
        pltpu.make_async_copy(buf, out_hbm.at[pl.ds(rbase, CHUNK)], sem).wait()

    for b in range(NBUF):
        start_gather(b, bufs[b], sg[b])

    def group(k, c):
        j0 = NBUF * k
        for b in range(NBUF):
            wait_gather(bufs[b], sg[b])
            start_store(j0 + b, bufs[b], ss[b])

        @pl.when(k < ngroups - 1)
        def _prefetch():
            for b in range(NBUF):
                wait_store(bufs[b], ss[b])
                start_gather(j0 + NBUF + b, bufs[b], sg[b])

        return c

    lax.fori_loop(0, ngroups, group, 0)
    for b in range(NBUF):
        wait_store(bufs[b], ss[b])


def kernel(input_ids, word_embeddings, ln_weight, ln_bias):
    B, T = input_ids.shape
    V, Dd = word_embeddings.shape
    N = B * T

    ln_table = pl.pallas_call(
        _ln_table_body,
        grid=(V // LN_BLK,),
        in_specs=[
            pl.BlockSpec((LN_BLK, Dd), lambda i: (i, 0)),
            pl.BlockSpec((Dd,), lambda i: (0,)),
            pl.BlockSpec((Dd,), lambda i: (0,)),
        ],
        out_specs=pl.BlockSpec((LN_BLK, Dd), lambda i: (i, 0)),
        out_shape=jax.ShapeDtypeStruct((V, Dd), jnp.float32),
        compiler_params=pltpu.CompilerParams(
            dimension_semantics=("parallel",),
        ),
    )(word_embeddings, ln_weight, ln_bias)

    ids_flat = input_ids.reshape(N)
    mesh = plsc.VectorSubcoreMesh(core_axis_name="c", subcore_axis_name="s")
    f = pl.kernel(
        _gather_body,
        mesh=mesh,
        out_type=jax.ShapeDtypeStruct((N, Dd), jnp.float32),
        scratch_types=(
            [pltpu.VMEM((N // NW,), jnp.int32)]
            + [pltpu.VMEM((CHUNK, Dd), jnp.float32) for _ in range(NBUF)]
            + [pltpu.SemaphoreType.DMA for _ in range(2 * NBUF)]
        ),
    )
    out = f(ids_flat, ln_table)
    return out.reshape(B, T, Dd)


# speedup vs baseline: 1.0006x; 1.0006x over previous
import jax
import jax.numpy as jnp
from jax import lax
from jax.experimental import pallas as pl
from jax.experimental.pallas import tpu as pltpu
from jax.experimental.pallas import tpu_sc as plsc

D = 128
EPS = 1e-12
NW = 32
CHUNK = 64
NBUF = 10
LN_BLK = 10000


def _ln_table_body(x_ref, w_ref, b_ref, o_ref):
    x = x_ref[...]
    ones = jnp.ones((x.shape[1], 1), jnp.float32)
    mean = jax.lax.dot(x, ones, preferred_element_type=jnp.float32) * (1.0 / x.shape[1])
    xc = x - mean
    var = jax.lax.dot(xc * xc, ones, preferred_element_type=jnp.float32) * (1.0 / x.shape[1])
    o_ref[...] = xc * lax.rsqrt(var + EPS) * w_ref[...] + b_ref[...]


def _gather_body(ids_hbm, table_hbm, out_hbm, idx_v, *rest):
    bufs = rest[:NBUF]
    sg = rest[NBUF:2 * NBUF]
    ss = rest[2 * NBUF:3 * NBUF]
    npw = ids_hbm.shape[0] // NW
    ncw = npw // CHUNK
    ngroups = ncw // NBUF
    wid = lax.axis_index("s") * 2 + lax.axis_index("c")
    rbase = pl.multiple_of(wid * npw, CHUNK)
    pltpu.sync_copy(ids_hbm.at[pl.ds(rbase, npw)], idx_v)

    def start_gather(j, buf, sem):
        coff = pl.multiple_of(j * CHUNK, CHUNK)
        pltpu.async_copy(table_hbm.at[idx_v.at[pl.ds(coff, CHUNK)]], buf, sem)

    def wait_gather(buf, sem):
        pltpu.make_async_copy(table_hbm.at[pl.ds(0, CHUNK)], buf, sem).wait()

    def start_store(j, buf, sem):
        ooff = pl.multiple_of(rbase + j * CHUNK, CHUNK)
        pltpu.async_copy(buf, out_hbm.at[pl.ds(ooff, CHUNK)], sem)

    def wait_store(buf, sem):
        pltpu.make_async_copy(buf, out_hbm.at[pl.ds(rbase, CHUNK)], sem).wait()

    for b in range(NBUF):
        start_gather(b, bufs[b], sg[b])

    def group(k, c):
        j0 = NBUF * k
        for b in range(NBUF):
            wait_gather(bufs[b], sg[b])
            start_store(j0 + b, bufs[b], ss[b])

        @pl.when(k < ngroups - 1)
        def _prefetch():
            for b in range(NBUF):
                wait_store(bufs[b], ss[b])
                start_gather(j0 + NBUF + b, bufs[b], sg[b])

        return c

    lax.fori_loop(0, ngroups, group, 0)
    for b in range(NBUF):
        wait_store(bufs[b], ss[b])


def kernel(input_ids, word_embeddings, ln_weight, ln_bias):
    B, T = input_ids.shape
    V, Dd = word_embeddings.shape
    N = B * T

    ln_table = pl.pallas_call(
        _ln_table_body,
        grid=(V // LN_BLK,),
        in_specs=[
            pl.BlockSpec((LN_BLK, Dd), lambda i: (i, 0)),
            pl.BlockSpec((Dd,), lambda i: (0,)),
            pl.BlockSpec((Dd,), lambda i: (0,)),
        ],
        out_specs=pl.BlockSpec((LN_BLK, Dd), lambda i: (i, 0)),
        out_shape=jax.ShapeDtypeStruct((V, Dd), jnp.float32),
        compiler_params=pltpu.CompilerParams(
            dimension_semantics=("parallel",),
        ),
    )(word_embeddings, ln_weight, ln_bias)

    ids_flat = input_ids.reshape(N)
    mesh = plsc.VectorSubcoreMesh(core_axis_name="c", subcore_axis_name="s")
    f = pl.kernel(
        _gather_body,
        mesh=mesh,
        out_type=jax.ShapeDtypeStruct((N, Dd), jnp.float32),
        scratch_types=(
            [pltpu.VMEM((N // NW,), jnp.int32)]
            + [pltpu.VMEM((CHUNK, Dd), jnp.float32) for _ in range(NBUF)]
            + [pltpu.SemaphoreType.DMA for _ in range(2 * NBUF)]
        ),
    )
    out = f(ids_flat, ln_table)
    return out.reshape(B, T, Dd)


# speedup vs baseline: 1.0065x; 1.0059x over previous
import jax
import jax.numpy as jnp
from jax import lax
from jax.experimental import pallas as pl
from jax.experimental.pallas import tpu as pltpu
from jax.experimental.pallas import tpu_sc as plsc

D = 128
EPS = 1e-12
NW = 32
CHUNK = 64
NBUF = 10
LN_BLK = 10000


def _ln_table_body(x_ref, w_ref, b_ref, o_ref):
    x = x_ref[...]
    mean = jnp.mean(x, axis=1, keepdims=True)
    xc = x - mean
    var = jnp.mean(xc * xc, axis=1, keepdims=True)
    o_ref[...] = xc * lax.rsqrt(var + EPS) * w_ref[...] + b_ref[...]


def _gather_body(ids_hbm, table_hbm, out_hbm, idx_v, *rest):
    bufs = rest[:NBUF]
    sg = rest[NBUF:2 * NBUF]
    ss = rest[2 * NBUF:3 * NBUF]
    npw = ids_hbm.shape[0] // NW
    ncw = npw // CHUNK
    ngroups = ncw // NBUF
    wid = lax.axis_index("s") * 2 + lax.axis_index("c")
    rbase = pl.multiple_of(wid * npw, CHUNK)
    pltpu.sync_copy(ids_hbm.at[pl.ds(rbase, npw)], idx_v)

    def start_gather(j, buf, sem):
        coff = pl.multiple_of(j * CHUNK, CHUNK)
        pltpu.async_copy(table_hbm.at[idx_v.at[pl.ds(coff, CHUNK)]], buf, sem)

    def wait_gather(buf, sem):
        pltpu.make_async_copy(table_hbm.at[pl.ds(0, CHUNK)], buf, sem).wait()

    def start_store(j, buf, sem):
        ooff = pl.multiple_of(rbase + j * CHUNK, CHUNK)
        pltpu.async_copy(buf, out_hbm.at[pl.ds(ooff, CHUNK)], sem)

    def wait_store(buf, sem):
        pltpu.make_async_copy(buf, out_hbm.at[pl.ds(rbase, CHUNK)], sem).wait()

    for b in range(NBUF):
        start_gather(b, bufs[b], sg[b])

    def group(k, c):
        j0 = NBUF * k
        for b in range(NBUF):
            wait_gather(bufs[b], sg[b])
            start_store(j0 + b, bufs[b], ss[b])

        @pl.when(k < ngroups - 1)
        def _prefetch():
            for b in range(NBUF):
                wait_store(bufs[b], ss[b])
                start_gather(j0 + NBUF + b, bufs[b], sg[b])

        return c

    lax.fori_loop(0, ngroups, group, 0)
    for b in range(NBUF):
        wait_store(bufs[b], ss[b])


def kernel(input_ids, word_embeddings, ln_weight, ln_bias):
    B, T = input_ids.shape
    V, Dd = word_embeddings.shape
    N = B * T

    ln_table = pl.pallas_call(
        _ln_table_body,
        grid=(V // LN_BLK,),
        in_specs=[
            pl.BlockSpec((LN_BLK, Dd), lambda i: (i, 0)),
            pl.BlockSpec((Dd,), lambda i: (0,)),
            pl.BlockSpec((Dd,), lambda i: (0,)),
        ],
        out_specs=pl.BlockSpec((LN_BLK, Dd), lambda i: (i, 0)),
        out_shape=jax.ShapeDtypeStruct((V, Dd), jnp.float32),
    )(word_embeddings, ln_weight, ln_bias)

    ids_flat = input_ids.reshape(N)
    mesh = plsc.VectorSubcoreMesh(core_axis_name="c", subcore_axis_name="s")
    f = pl.kernel(
        _gather_body,
        mesh=mesh,
        out_type=jax.ShapeDtypeStruct((N, Dd), jnp.float32),
        scratch_types=(
            [pltpu.VMEM((N // NW,), jnp.int32)]
            + [pltpu.VMEM((CHUNK, Dd), jnp.float32) for _ in range(NBUF)]
            + [pltpu.SemaphoreType.DMA for _ in range(2 * NBUF)]
        ),
    )
    out = f(ids_flat, ln_table)
    return out.reshape(B, T, Dd)
